# Initial kernel scaffold; baseline (speedup 1.0000x reference)
import functools

import jax
import jax.numpy as jnp
from jax import lax
from jax.experimental import pallas as pl
from jax.experimental.pallas import tpu as pltpu
from jax.experimental.pallas import tpu_sc as plsc

NGENE = 4762
NCELL = 847
DIM = 256
NE_ENC = 150000
NE_DEC = 200000

NTILE = 32
ECHUNK = 1200
NCHUNK = NE_ENC // ECHUNK
VPC = ECHUNK // 16

R1 = 27
R2 = 149
W1 = R1 * NGENE
W2 = R2 * NCELL
ACCW = 128576

EDEC_PAD = 200704
EPT = EDEC_PAD // NTILE


def _mesh():
  return plsc.VectorSubcoreMesh(core_axis_name="c", subcore_axis_name="s")


def _wid():
  return lax.axis_index("s") * 2 + lax.axis_index("c")


@functools.partial(
    pl.kernel,
    out_type=[
        jax.ShapeDtypeStruct((NTILE, W1), jnp.float32),
        jax.ShapeDtypeStruct((NTILE, W2), jnp.float32),
    ],
    mesh=_mesh(),
    scratch_types=[
        pltpu.VMEM((ACCW,), jnp.float32),
        pltpu.VMEM((ECHUNK,), jnp.int32),
        pltpu.VMEM((ECHUNK,), jnp.int32),
    ],
)
def _adj(s1, d1, s2, d2, a1, a2, acc, sbuf, dbuf):
  wid = _wid()

  def phase(src_hbm, dst_hbm, rows, cols, words, out):
    lo = wid * rows

    def zero(i, _):
      acc[pl.ds(i * 16, 16)] = jnp.zeros((16,), jnp.float32)
      return 0

    lax.fori_loop(0, ACCW // 16, zero, 0)

    def chunk(k, _):
      pltpu.sync_copy(src_hbm.at[pl.ds(k * ECHUNK, ECHUNK)], sbuf)
      pltpu.sync_copy(dst_hbm.at[pl.ds(k * ECHUNK, ECHUNK)], dbuf)

      def vec(j, _):
        sv = sbuf[pl.ds(j * 16, 16)]
        dv = dbuf[pl.ds(j * 16, 16)]
        m = (dv >= lo) & (dv < lo + rows)
        idx = jnp.where(m, (dv - lo) * cols + sv, 0)
        ones = jnp.where(m, 1.0, 0.0).astype(jnp.float32)
        plsc.addupdate_scatter(acc, [idx], ones)
        return 0

      lax.fori_loop(0, VPC, vec, 0)
      return 0

    lax.fori_loop(0, NCHUNK, chunk, 0)
    pltpu.sync_copy(acc.at[pl.ds(0, words)], out.at[wid])

  phase(s1, d1, R1, NGENE, W1, a1)
  phase(s2, d2, R2, NCELL, W2, a2)


def _conv_body(a_ref, x_ref, w_ref, b_ref, wp_ref, bp_ref, h_ref, p_ref):
  a = a_ref[...]
  din = jnp.maximum(jnp.sum(a, axis=1, keepdims=True), 1.0)
  dout = jnp.maximum(jnp.sum(a, axis=0, keepdims=True), 1.0)
  a = a * lax.rsqrt(dout)
  agg = jnp.dot(a, x_ref[...], preferred_element_type=jnp.float32,
                precision=lax.Precision.HIGHEST)
  agg = agg * lax.rsqrt(din)
  h = jnp.dot(agg, w_ref[...], preferred_element_type=jnp.float32,
              precision=lax.Precision.HIGHEST) + b_ref[...]
  h = jnp.maximum(h, 0.0)
  h_ref[...] = h
  p_ref[...] = jnp.sum(h * wp_ref[...], axis=1, keepdims=True) + bp_ref[0]


def _conv(a, x, w, b, wp, bp):
  n_dst = a.shape[0]
  return pl.pallas_call(
      _conv_body,
      out_shape=(
          jax.ShapeDtypeStruct((n_dst, DIM), jnp.float32),
          jax.ShapeDtypeStruct((n_dst, 1), jnp.float32),
      ),
      in_specs=[
          pl.BlockSpec(memory_space=pltpu.VMEM),
          pl.BlockSpec(memory_space=pltpu.VMEM),
          pl.BlockSpec(memory_space=pltpu.VMEM),
          pl.BlockSpec(memory_space=pltpu.VMEM),
          pl.BlockSpec(memory_space=pltpu.VMEM),
          pl.BlockSpec(memory_space=pltpu.SMEM),
      ],
      out_specs=(
          pl.BlockSpec(memory_space=pltpu.VMEM),
          pl.BlockSpec(memory_space=pltpu.VMEM),
      ),
  )(a, x, w, b, wp, bp)


@functools.partial(
    pl.kernel,
    out_type=jax.ShapeDtypeStruct((EDEC_PAD,), jnp.float32),
    mesh=_mesh(),
    scratch_types=[
        pltpu.VMEM((NGENE,), jnp.float32),
        pltpu.VMEM((NCELL,), jnp.float32),
        pltpu.VMEM((EPT,), jnp.int32),
        pltpu.VMEM((EPT,), jnp.int32),
        pltpu.VMEM((EPT,), jnp.float32),
    ],
)
def _dec(pg_hbm, pc_hbm, src_hbm, dst_hbm, out_hbm, pg, pc, sbuf, dbuf, obuf):
  wid = _wid()
  base = wid * EPT
  pltpu.sync_copy(pg_hbm, pg)
  pltpu.sync_copy(pc_hbm, pc)
  pltpu.sync_copy(src_hbm.at[pl.ds(base, EPT)], sbuf)
  pltpu.sync_copy(dst_hbm.at[pl.ds(base, EPT)], dbuf)

  def vec(j, _):
    sv = sbuf[pl.ds(j * 16, 16)]
    dv = dbuf[pl.ds(j * 16, 16)]
    g = plsc.load_gather(pg, [sv])
    c = plsc.load_gather(pc, [dv])
    obuf[pl.ds(j * 16, 16)] = g + c
    return 0

  lax.fori_loop(0, EPT // 16, vec, 0)
  pltpu.sync_copy(obuf, out_hbm.at[pl.ds(base, EPT)])


def kernel(enc_g2c_src, enc_g2c_dst, enc_c2g_src, enc_c2g_dst,
           dec_src, dec_dst,
           gene_emb, cell_emb, W_g2c, b_g2c, W_c2g, b_c2g, Wp, bp):
  i32 = jnp.int32
  a1f, a2f = _adj(enc_g2c_src.astype(i32), enc_g2c_dst.astype(i32),
                  enc_c2g_src.astype(i32), enc_c2g_dst.astype(i32))
  a1 = a1f.reshape(NTILE * R1, NGENE)[:NCELL]
  a2 = a2f.reshape(NTILE * R2, NCELL)[:NGENE]

  wp_g = Wp[:DIM, 0].reshape(1, DIM)
  wp_c = Wp[DIM:, 0].reshape(1, DIM)
  h_gene, p_gene = _conv(a2, cell_emb, W_c2g, b_c2g.reshape(1, DIM), wp_g, bp)
  h_cell, p_cell = _conv(a1, gene_emb, W_g2c, b_g2c.reshape(1, DIM), wp_c,
                         jnp.zeros_like(bp))

  pad = jnp.zeros((EDEC_PAD - NE_DEC,), i32)
  srcp = jnp.concatenate([dec_src.astype(i32), pad])
  dstp = jnp.concatenate([dec_dst.astype(i32), pad])
  score = _dec(p_gene.reshape(-1), p_cell.reshape(-1), srcp, dstp)
  score = score[:NE_DEC].reshape(NE_DEC, 1)
  return score, h_gene, h_cell


# speedup vs baseline: 2.2916x; 2.2916x over previous
import functools

import jax
import jax.numpy as jnp
from jax import lax
from jax.experimental import pallas as pl
from jax.experimental.pallas import tpu as pltpu
from jax.experimental.pallas import tpu_sc as plsc

NGENE = 4762
NCELL = 847
DIM = 256
NE_ENC = 150000
NE_DEC = 200000

NTILE = 32
ECHUNK = 400
NCHUNK = NE_ENC // ECHUNK
VPC = ECHUNK // 16

R1 = 27
R2 = 149
W1 = R1 * NGENE
W2 = R2 * NCELL
W1P = 128576
W2P = 126208
ACCW = 128576

EDEC_PAD = 200704
EPT = EDEC_PAD // NTILE


def _mesh():
  return plsc.VectorSubcoreMesh(core_axis_name="c", subcore_axis_name="s")


def _wid():
  return lax.axis_index("s") * 2 + lax.axis_index("c")


@functools.partial(
    pl.kernel,
    out_type=[
        jax.ShapeDtypeStruct((NTILE * W1P,), jnp.float32),
        jax.ShapeDtypeStruct((NTILE * W2P,), jnp.float32),
    ],
    mesh=_mesh(),
    scratch_types=[
        pltpu.VMEM((ACCW,), jnp.float32),
        pltpu.VMEM((ECHUNK,), jnp.int32),
        pltpu.VMEM((ECHUNK,), jnp.int32),
    ],
    compiler_params=pltpu.CompilerParams(needs_layout_passes=False),
)
def _adj(s1, d1, s2, d2, a1, a2, acc, sbuf, dbuf):
  wid = _wid()

  def phase(src_hbm, dst_hbm, rows, cols, stride, out):
    lo = wid * rows

    def zero(i, _):
      acc[pl.ds(i * 16, 16)] = jnp.zeros((16,), jnp.float32)
      return 0

    lax.fori_loop(0, ACCW // 16, zero, 0)

    def chunk(k, _):
      pltpu.sync_copy(src_hbm.at[pl.ds(k * ECHUNK, ECHUNK)], sbuf)
      pltpu.sync_copy(dst_hbm.at[pl.ds(k * ECHUNK, ECHUNK)], dbuf)

      def vec(j, _):
        sv = sbuf[pl.ds(j * 16, 16)]
        dv = dbuf[pl.ds(j * 16, 16)]
        m = (dv >= lo) & (dv < lo + rows)
        idx = jnp.where(m, (dv - lo) * cols + sv, 0)
        ones = jnp.where(m, 1.0, 0.0).astype(jnp.float32)
        plsc.addupdate_scatter(acc, [idx], ones)
        return 0

      lax.fori_loop(0, VPC, vec, 0)
      return 0

    lax.fori_loop(0, NCHUNK, chunk, 0)
    pltpu.sync_copy(acc.at[pl.ds(0, stride)],
                    out.at[pl.ds(wid * stride, stride)])

  phase(s1, d1, R1, NGENE, W1P, a1)
  phase(s2, d2, R2, NCELL, W2P, a2)


KB = 128


def _conv_body(n_src, nkb, a_ref, x_ref, w_ref, b_ref, wp_ref, bp_ref,
               h_ref, p_ref, agg_ref, din_ref):
  k = pl.program_id(0)

  @pl.when(k == 0)
  def _init():
    agg_ref[...] = jnp.zeros_like(agg_ref)
    din_ref[...] = jnp.zeros_like(din_ref)

  a = a_ref[...]
  col = k * KB + lax.broadcasted_iota(jnp.int32, (1, KB), 1)
  a = jnp.where(col < n_src, a, 0.0)
  din_ref[...] += jnp.sum(a, axis=1, keepdims=True)
  dout = jnp.maximum(jnp.sum(a, axis=0, keepdims=True), 1.0)
  rs = jnp.transpose(lax.rsqrt(dout))
  row = k * KB + lax.broadcasted_iota(jnp.int32, (KB, 1), 0)
  xs = jnp.where(row < n_src, x_ref[...] * rs, 0.0)
  agg_ref[...] += jnp.dot(a, xs, preferred_element_type=jnp.float32,
                          precision=lax.Precision.HIGHEST)

  @pl.when(k == nkb - 1)
  def _fin():
    agg = agg_ref[...] * lax.rsqrt(jnp.maximum(din_ref[...], 1.0))
    h = jnp.dot(agg, w_ref[...], preferred_element_type=jnp.float32,
                precision=lax.Precision.HIGHEST) + b_ref[...]
    h = jnp.maximum(h, 0.0)
    h_ref[...] = h
    p_ref[...] = jnp.sum(h * wp_ref[...], axis=1, keepdims=True) + bp_ref[0]


def _conv(a, x, w, b, wp, bp):
  n_dst, n_src = a.shape
  nkb = (n_src + KB - 1) // KB
  return pl.pallas_call(
      functools.partial(_conv_body, n_src, nkb),
      grid=(nkb,),
      out_shape=(
          jax.ShapeDtypeStruct((n_dst, DIM), jnp.float32),
          jax.ShapeDtypeStruct((n_dst, 1), jnp.float32),
      ),
      in_specs=[
          pl.BlockSpec((n_dst, KB), lambda k: (0, k)),
          pl.BlockSpec((KB, DIM), lambda k: (k, 0)),
          pl.BlockSpec((DIM, DIM), lambda k: (0, 0)),
          pl.BlockSpec((1, DIM), lambda k: (0, 0)),
          pl.BlockSpec((1, DIM), lambda k: (0, 0)),
          pl.BlockSpec(memory_space=pltpu.SMEM),
      ],
      out_specs=(
          pl.BlockSpec((n_dst, DIM), lambda k: (0, 0)),
          pl.BlockSpec((n_dst, 1), lambda k: (0, 0)),
      ),
      scratch_shapes=[
          pltpu.VMEM((n_dst, DIM), jnp.float32),
          pltpu.VMEM((n_dst, 1), jnp.float32),
      ],
  )(a, x, w, b, wp, bp)


@functools.partial(
    pl.kernel,
    out_type=jax.ShapeDtypeStruct((EDEC_PAD,), jnp.float32),
    mesh=_mesh(),
    scratch_types=[
        pltpu.VMEM((NGENE,), jnp.float32),
        pltpu.VMEM((NCELL,), jnp.float32),
        pltpu.VMEM((EPT,), jnp.int32),
        pltpu.VMEM((EPT,), jnp.int32),
        pltpu.VMEM((EPT,), jnp.float32),
    ],
    compiler_params=pltpu.CompilerParams(needs_layout_passes=False),
)
def _dec(pg_hbm, pc_hbm, src_hbm, dst_hbm, out_hbm, pg, pc, sbuf, dbuf, obuf):
  wid = _wid()
  base = wid * EPT
  pltpu.sync_copy(pg_hbm, pg)
  pltpu.sync_copy(pc_hbm, pc)
  pltpu.sync_copy(src_hbm.at[pl.ds(base, EPT)], sbuf)
  pltpu.sync_copy(dst_hbm.at[pl.ds(base, EPT)], dbuf)

  def vec(j, _):
    sv = sbuf[pl.ds(j * 16, 16)]
    dv = dbuf[pl.ds(j * 16, 16)]
    g = plsc.load_gather(pg, [sv])
    c = plsc.load_gather(pc, [dv])
    obuf[pl.ds(j * 16, 16)] = g + c
    return 0

  lax.fori_loop(0, EPT // 16, vec, 0)
  pltpu.sync_copy(obuf, out_hbm.at[pl.ds(base, EPT)])


def kernel(enc_g2c_src, enc_g2c_dst, enc_c2g_src, enc_c2g_dst,
           dec_src, dec_dst,
           gene_emb, cell_emb, W_g2c, b_g2c, W_c2g, b_c2g, Wp, bp):
  i32 = jnp.int32
  a1f, a2f = _adj(enc_g2c_src.astype(i32), enc_g2c_dst.astype(i32),
                  enc_c2g_src.astype(i32), enc_c2g_dst.astype(i32))
  a1 = a1f.reshape(NTILE, W1P)[:, :W1].reshape(NTILE * R1, NGENE)
  a2 = a2f.reshape(NTILE, W2P)[:, :W2].reshape(NTILE * R2, NCELL)

  wp_g = Wp[:DIM, 0].reshape(1, DIM)
  wp_c = Wp[DIM:, 0].reshape(1, DIM)
  h_gene, p_gene = _conv(a2, cell_emb, W_c2g, b_c2g.reshape(1, DIM), wp_g, bp)
  h_cell, p_cell = _conv(a1, gene_emb, W_g2c, b_g2c.reshape(1, DIM), wp_c,
                         jnp.zeros_like(bp))
  h_gene = h_gene[:NGENE]
  h_cell = h_cell[:NCELL]
  p_gene = p_gene[:NGENE]
  p_cell = p_cell[:NCELL]

  pad = jnp.zeros((EDEC_PAD - NE_DEC,), i32)
  srcp = jnp.concatenate([dec_src.astype(i32), pad])
  dstp = jnp.concatenate([dec_dst.astype(i32), pad])
  score = _dec(p_gene.reshape(-1), p_cell.reshape(-1), srcp, dstp)
  score = score[:NE_DEC].reshape(NE_DEC, 1)
  return score, h_gene, h_cell


# speedup vs baseline: 2.8727x; 1.2536x over previous
import functools

import jax
import jax.numpy as jnp
from jax import lax
from jax.experimental import pallas as pl
from jax.experimental.pallas import tpu as pltpu
from jax.experimental.pallas import tpu_sc as plsc

NGENE = 4762
NCELL = 847
DIM = 256
NE_ENC = 150000
NE_DEC = 200000

NTILE = 32
ECHUNK = 400
NCHUNK = NE_ENC // ECHUNK
VPC = ECHUNK // 16

R1 = 27
R2 = 149
W1 = R1 * NGENE
W2 = R2 * NCELL
W1P = 128576
W2P = 126208
ACCW = 128576

EDEC_PAD = 200704
EPT = EDEC_PAD // NTILE


def _mesh():
  return plsc.VectorSubcoreMesh(core_axis_name="c", subcore_axis_name="s")


def _wid():
  return lax.axis_index("s") * 2 + lax.axis_index("c")


@functools.partial(
    pl.kernel,
    out_type=[
        jax.ShapeDtypeStruct((NTILE * W1P,), jnp.float32),
        jax.ShapeDtypeStruct((NTILE * W2P,), jnp.float32),
    ],
    mesh=_mesh(),
    scratch_types=[
        pltpu.VMEM((ACCW,), jnp.float32),
        pltpu.VMEM((ECHUNK,), jnp.int32),
        pltpu.VMEM((ECHUNK,), jnp.int32),
    ],
    compiler_params=pltpu.CompilerParams(needs_layout_passes=False),
)
def _adj(s1, d1, s2, d2, a1, a2, acc, sbuf, dbuf):
  wid = _wid()

  ones = jnp.ones((16,), jnp.float32)

  def phase(src_hbm, dst_hbm, rows, cols, stride, out):
    lo = wid * rows

    def zero(i, _):
      acc[pl.ds(i * 16, 16)] = jnp.zeros((16,), jnp.float32)
      return 0

    lax.fori_loop(0, ACCW // 16, zero, 0)

    def chunk(k, _):
      pltpu.sync_copy(src_hbm.at[pl.ds(k * ECHUNK, ECHUNK)], sbuf)
      pltpu.sync_copy(dst_hbm.at[pl.ds(k * ECHUNK, ECHUNK)], dbuf)

      def vec(j, _):
        sv = sbuf[pl.ds(j * 16, 16)]
        dv = dbuf[pl.ds(j * 16, 16)]
        m = (dv >= lo) & (dv < lo + rows)
        idx = (dv - lo) * cols + sv
        plsc.addupdate_scatter(acc, [idx], ones, mask=m)
        return 0

      lax.fori_loop(0, VPC, vec, 0)
      return 0

    lax.fori_loop(0, NCHUNK, chunk, 0)
    pltpu.sync_copy(acc.at[pl.ds(0, stride)],
                    out.at[pl.ds(wid * stride, stride)])

  phase(s1, d1, R1, NGENE, W1P, a1)
  phase(s2, d2, R2, NCELL, W2P, a2)


KB = 128


def _conv_body(n_src, nkb, a_ref, x_ref, w_ref, b_ref, wp_ref, bp_ref,
               h_ref, p_ref, agg_ref, din_ref):
  k = pl.program_id(0)

  @pl.when(k == 0)
  def _init():
    agg_ref[...] = jnp.zeros_like(agg_ref)
    din_ref[...] = jnp.zeros_like(din_ref)

  a = a_ref[...]
  col = k * KB + lax.broadcasted_iota(jnp.int32, (1, KB), 1)
  a = jnp.where(col < n_src, a, 0.0)
  din_ref[...] += jnp.sum(a, axis=1, keepdims=True)
  dout = jnp.maximum(jnp.sum(a, axis=0, keepdims=True), 1.0)
  rs = jnp.transpose(lax.rsqrt(dout))
  row = k * KB + lax.broadcasted_iota(jnp.int32, (KB, 1), 0)
  xs = jnp.where(row < n_src, x_ref[...] * rs, 0.0)
  agg_ref[...] += jnp.dot(a, xs, preferred_element_type=jnp.float32,
                          precision=lax.Precision.DEFAULT)

  @pl.when(k == nkb - 1)
  def _fin():
    agg = agg_ref[...] * lax.rsqrt(jnp.maximum(din_ref[...], 1.0))
    h = jnp.dot(agg, w_ref[...], preferred_element_type=jnp.float32,
                precision=lax.Precision.DEFAULT) + b_ref[...]
    h = jnp.maximum(h, 0.0)
    h_ref[...] = h
    p_ref[...] = jnp.sum(h * wp_ref[...], axis=1, keepdims=True) + bp_ref[0]


def _conv(a, x, w, b, wp, bp):
  n_dst, n_src = a.shape
  nkb = (n_src + KB - 1) // KB
  return pl.pallas_call(
      functools.partial(_conv_body, n_src, nkb),
      grid=(nkb,),
      out_shape=(
          jax.ShapeDtypeStruct((n_dst, DIM), jnp.float32),
          jax.ShapeDtypeStruct((n_dst, 1), jnp.float32),
      ),
      in_specs=[
          pl.BlockSpec((n_dst, KB), lambda k: (0, k)),
          pl.BlockSpec((KB, DIM), lambda k: (k, 0)),
          pl.BlockSpec((DIM, DIM), lambda k: (0, 0)),
          pl.BlockSpec((1, DIM), lambda k: (0, 0)),
          pl.BlockSpec((1, DIM), lambda k: (0, 0)),
          pl.BlockSpec(memory_space=pltpu.SMEM),
      ],
      out_specs=(
          pl.BlockSpec((n_dst, DIM), lambda k: (0, 0)),
          pl.BlockSpec((n_dst, 1), lambda k: (0, 0)),
      ),
      scratch_shapes=[
          pltpu.VMEM((n_dst, DIM), jnp.float32),
          pltpu.VMEM((n_dst, 1), jnp.float32),
      ],
  )(a, x, w, b, wp, bp)


@functools.partial(
    pl.kernel,
    out_type=jax.ShapeDtypeStruct((EDEC_PAD,), jnp.float32),
    mesh=_mesh(),
    scratch_types=[
        pltpu.VMEM((NGENE,), jnp.float32),
        pltpu.VMEM((NCELL,), jnp.float32),
        pltpu.VMEM((EPT,), jnp.int32),
        pltpu.VMEM((EPT,), jnp.int32),
        pltpu.VMEM((EPT,), jnp.float32),
    ],
    compiler_params=pltpu.CompilerParams(needs_layout_passes=False),
)
def _dec(pg_hbm, pc_hbm, src_hbm, dst_hbm, out_hbm, pg, pc, sbuf, dbuf, obuf):
  wid = _wid()
  base = wid * EPT
  pltpu.sync_copy(pg_hbm, pg)
  pltpu.sync_copy(pc_hbm, pc)
  pltpu.sync_copy(src_hbm.at[pl.ds(base, EPT)], sbuf)
  pltpu.sync_copy(dst_hbm.at[pl.ds(base, EPT)], dbuf)

  def vec(j, _):
    sv = sbuf[pl.ds(j * 16, 16)]
    dv = dbuf[pl.ds(j * 16, 16)]
    g = plsc.load_gather(pg, [sv])
    c = plsc.load_gather(pc, [dv])
    obuf[pl.ds(j * 16, 16)] = g + c
    return 0

  lax.fori_loop(0, EPT // 16, vec, 0)
  pltpu.sync_copy(obuf, out_hbm.at[pl.ds(base, EPT)])


def kernel(enc_g2c_src, enc_g2c_dst, enc_c2g_src, enc_c2g_dst,
           dec_src, dec_dst,
           gene_emb, cell_emb, W_g2c, b_g2c, W_c2g, b_c2g, Wp, bp):
  i32 = jnp.int32
  a1f, a2f = _adj(enc_g2c_src.astype(i32), enc_g2c_dst.astype(i32),
                  enc_c2g_src.astype(i32), enc_c2g_dst.astype(i32))
  a1 = a1f.reshape(NTILE, W1P)[:, :W1].reshape(NTILE * R1, NGENE)
  a2 = a2f.reshape(NTILE, W2P)[:, :W2].reshape(NTILE * R2, NCELL)

  wp_g = Wp[:DIM, 0].reshape(1, DIM)
  wp_c = Wp[DIM:, 0].reshape(1, DIM)
  h_gene, p_gene = _conv(a2, cell_emb, W_c2g, b_c2g.reshape(1, DIM), wp_g, bp)
  h_cell, p_cell = _conv(a1, gene_emb, W_g2c, b_g2c.reshape(1, DIM), wp_c,
                         jnp.zeros_like(bp))
  h_gene = h_gene[:NGENE]
  h_cell = h_cell[:NCELL]
  p_gene = p_gene[:NGENE]
  p_cell = p_cell[:NCELL]

  pad = jnp.zeros((EDEC_PAD - NE_DEC,), i32)
  srcp = jnp.concatenate([dec_src.astype(i32), pad])
  dstp = jnp.concatenate([dec_dst.astype(i32), pad])
  score = _dec(p_gene.reshape(-1), p_cell.reshape(-1), srcp, dstp)
  score = score[:NE_DEC].reshape(NE_DEC, 1)
  return score, h_gene, h_cell
